# baseline (speedup 1.0000x reference)
import functools

import jax
import jax.numpy as jnp
from jax import lax
from jax.experimental import pallas as pl
from jax.experimental.pallas import tpu as pltpu
from jax.experimental.pallas import tpu_sc as plsc

T = 2048
H = 1024
E = 512
NR = 64
NE = NR + 1
TB = 256
NB = T // TB
BLK = 64
NRB = 127
SBLK = 128
NTAB = NRB
NROW = NRB * BLK
NW = 32
TPW = T // NW
CH = 16
NCH = TPW // CH


def _route_meta_body(x_ref, rw_ref,
                     s0_ref, s1_ref, g0_ref, g1_ref, be_ref, nbt_ref,
                     i1_s, i2_s, r0_s, r1_s, c0_s, c1_s):
    i = pl.program_id(0)

    @pl.when(i < NB)
    def _router():
        x = x_ref[...]
        logits = lax.dot_general(x, rw_ref[...], (((1,), (1,)), ((), ())),
                                 preferred_element_type=jnp.float32)
        m = jnp.max(logits, axis=-1, keepdims=True)
        ex = jnp.exp(logits - m)
        probs = ex / jnp.sum(ex, axis=-1, keepdims=True)
        idx = lax.broadcasted_iota(jnp.int32, probs.shape, 1)
        m1 = jnp.max(probs, axis=-1, keepdims=True)
        i1 = jnp.min(jnp.where(probs == m1, idx, NR), axis=-1, keepdims=True)
        masked = jnp.where(idx == i1, -1.0, probs)
        m2 = jnp.max(masked, axis=-1, keepdims=True)
        i2 = jnp.min(jnp.where(masked == m2, idx, NR), axis=-1,
                     keepdims=True)
        off = i * TB
        g0_ref[pl.ds(off, TB), :] = jnp.broadcast_to(m1, (TB, 16))
        g1_ref[pl.ds(off, TB), :] = jnp.broadcast_to(m2, (TB, 16))
        i1_s[pl.ds(off, TB), :] = i1
        i2_s[pl.ds(off, TB), :] = i2
        o1 = (i1 == idx).astype(jnp.float32)
        o2 = (i2 == idx).astype(jnp.float32)
        r_i = lax.broadcasted_iota(jnp.int32, (TB, TB), 0)
        c_i = lax.broadcasted_iota(jnp.int32, (TB, TB), 1)
        tril = (r_i > c_i).astype(jnp.float32)
        cum1 = lax.dot_general(tril, o1, (((1,), (0,)), ((), ())),
                               preferred_element_type=jnp.float32)
        cum2 = lax.dot_general(tril, o2, (((1,), (0,)), ((), ())),
                               preferred_element_type=jnp.float32)
        r0_s[pl.ds(off, TB), :] = jnp.sum(
            cum1 * o1, axis=-1, keepdims=True).astype(jnp.int32)
        r1_s[pl.ds(off, TB), :] = jnp.sum(
            cum2 * o2, axis=-1, keepdims=True).astype(jnp.int32)
        rowsel = lax.broadcasted_iota(jnp.int32, (NB, NR), 0) == i
        c0_s[...] = jnp.where(rowsel, jnp.sum(o1, axis=0, keepdims=True),
                              c0_s[...])
        c1_s[...] = jnp.where(rowsel, jnp.sum(o2, axis=0, keepdims=True),
                              c1_s[...])

    @pl.when(i == NB)
    def _meta():
        c0 = c0_s[...]
        c1 = c1_s[...]
        r8 = lax.broadcasted_iota(jnp.int32, (NB, NB), 0)
        c8 = lax.broadcasted_iota(jnp.int32, (NB, NB), 1)
        tril8 = (r8 > c8).astype(jnp.float32)
        carry0 = lax.dot_general(tril8, c0, (((1,), (0,)), ((), ())),
                                 preferred_element_type=jnp.float32)
        carry1 = lax.dot_general(tril8, c1, (((1,), (0,)), ((), ())),
                                 preferred_element_type=jnp.float32)
        cnt0 = jnp.sum(c0, axis=0, keepdims=True)
        cnt1 = jnp.sum(c1, axis=0, keepdims=True)
        ctot = (cnt0 + cnt1).astype(jnp.int32)
        nb = (ctot + (BLK - 1)) // BLK
        nbf = nb.astype(jnp.float32)
        rE = lax.broadcasted_iota(jnp.int32, (NR, NR), 0)
        cE = lax.broadcasted_iota(jnp.int32, (NR, NR), 1)
        triu = (rE < cE).astype(jnp.float32)
        bc_excl = lax.dot_general(nbf, triu, (((1,), (0,)), ((), ())),
                                  preferred_element_type=jnp.float32)
        bc_incl = bc_excl + nbf
        nbt_ref[...] = jnp.sum(nb, axis=-1, keepdims=True)
        tb_of = lax.broadcasted_iota(jnp.int32, (T, NB), 0) // TB
        sel = (tb_of == lax.broadcasted_iota(jnp.int32, (T, NB), 1)).astype(
            jnp.float32)
        car0 = lax.dot_general(sel, carry0, (((1,), (0,)), ((), ())),
                               preferred_element_type=jnp.float32)
        car1 = lax.dot_general(sel, carry1, (((1,), (0,)), ((), ())),
                               preferred_element_type=jnp.float32)
        ie = lax.broadcasted_iota(jnp.int32, (T, NR), 1)
        o1 = (i1_s[...] == ie).astype(jnp.float32)
        o2 = (i2_s[...] == ie).astype(jnp.float32)
        base0 = jnp.sum(o1 * (BLK * bc_excl + car0), axis=-1, keepdims=True)
        base1 = jnp.sum(o2 * (BLK * bc_excl + cnt0 + car1), axis=-1,
                        keepdims=True)
        s0_ref[...] = base0.astype(jnp.int32) + r0_s[...]
        s1_ref[...] = base1.astype(jnp.int32) + r1_s[...]
        bi = lax.broadcasted_iota(jnp.int32, (2 * BLK, NR), 0).astype(
            jnp.float32)
        be = jnp.sum((bi >= bc_incl).astype(jnp.float32), axis=-1,
                     keepdims=True)
        be_ref[...] = be.astype(jnp.int32)


def _run_route_meta(x, router_W, *, interpret=False):
    full = lambda i: (0, 0)
    return pl.pallas_call(
        _route_meta_body,
        grid=(NB + 1,),
        in_specs=[
            pl.BlockSpec((TB, H), lambda i: (jnp.minimum(i, NB - 1), 0)),
            pl.BlockSpec((NR, H), full),
        ],
        out_specs=[
            pl.BlockSpec((T, 1), full),
            pl.BlockSpec((T, 1), full),
            pl.BlockSpec((T, 16), full),
            pl.BlockSpec((T, 16), full),
            pl.BlockSpec((2 * BLK, 1), full),
            pl.BlockSpec((1, 1), full),
        ],
        out_shape=[
            jax.ShapeDtypeStruct((T, 1), jnp.int32),
            jax.ShapeDtypeStruct((T, 1), jnp.int32),
            jax.ShapeDtypeStruct((T, 16), jnp.float32),
            jax.ShapeDtypeStruct((T, 16), jnp.float32),
            jax.ShapeDtypeStruct((2 * BLK, 1), jnp.int32),
            jax.ShapeDtypeStruct((1, 1), jnp.int32),
        ],
        scratch_shapes=[
            pltpu.VMEM((T, 1), jnp.int32),
            pltpu.VMEM((T, 1), jnp.int32),
            pltpu.VMEM((T, 1), jnp.int32),
            pltpu.VMEM((T, 1), jnp.int32),
            pltpu.VMEM((NB, NR), jnp.float32),
            pltpu.VMEM((NB, NR), jnp.float32),
        ],
        interpret=interpret,
    )(x, router_W)


def _dispatch_body(x_hbm, sidx_hbm, xs_hbm,
                   rbuf0, rbuf1, idx0, idx1, lsem, sem):
    wid = lax.axis_index("s") * 2 + lax.axis_index("c")
    rbufs = (rbuf0, rbuf1)
    idxs = (idx0, idx1)
    base = wid * TPW
    ld0 = pltpu.async_copy(x_hbm.at[pl.ds(base, CH)], rbuf0, lsem)
    loads = [ld0]
    for c in range(NCH):
        if c + 1 < NCH:
            loads.append(pltpu.async_copy(
                x_hbm.at[pl.ds(base + (c + 1) * CH, CH)],
                rbufs[(c + 1) % 2], lsem))
        j = (wid * NCH + c) * 2
        pltpu.sync_copy(sidx_hbm.at[j], idxs[0])
        pltpu.sync_copy(sidx_hbm.at[j + 1], idxs[1])
        loads[c].wait()
        s0 = pltpu.async_copy(rbufs[c % 2], xs_hbm.at[idxs[0]], sem)
        s1 = pltpu.async_copy(rbufs[c % 2], xs_hbm.at[idxs[1]], sem)
        s0.wait()
        s1.wait()


def _run_dispatch(x, sidx, *, interpret=False):
    mesh = plsc.VectorSubcoreMesh(core_axis_name="c", subcore_axis_name="s")
    f = pl.kernel(
        _dispatch_body,
        out_type=jax.ShapeDtypeStruct((NROW, H), jnp.float32),
        mesh=mesh,
        scratch_types=[
            pltpu.VMEM((CH, H), jnp.float32),
            pltpu.VMEM((CH, H), jnp.float32),
            pltpu.VMEM((CH,), jnp.int32),
            pltpu.VMEM((CH,), jnp.int32),
            pltpu.SemaphoreType.DMA,
            pltpu.SemaphoreType.DMA,
        ],
        interpret=interpret,
    )
    return f(x, sidx)


def _gelu(u):
    return 0.5 * u * (1.0 + lax.erf(u * (2.0 ** -0.5)))


def _expert_block(xb, wu, wd):
    up = lax.dot_general(xb.astype(jnp.bfloat16), wu.astype(jnp.bfloat16),
                         (((1,), (1,)), ((), ())),
                         preferred_element_type=jnp.float32)
    act = _gelu(up)
    return lax.dot_general(act.astype(jnp.bfloat16),
                           wd.astype(jnp.bfloat16),
                           (((1,), (1,)), ((), ())),
                           preferred_element_type=jnp.float32)


def _gmm_body(sp_ref, xs_ref, ru_ref, rd_ref, out_ref):
    i = pl.program_id(0)

    @pl.when(i < sp_ref[0])
    def _():
        out_ref[...] = _expert_block(xs_ref[...], ru_ref[0], rd_ref[0])


def _shared_body(x_ref, su_ref, sd_ref, out_ref):
    out_ref[...] = _expert_block(x_ref[...], su_ref[0], sd_ref[0])


def _run_shared(x, su, sd, *, interpret=False):
    return pl.pallas_call(
        _shared_body,
        grid=(T // SBLK,),
        in_specs=[
            pl.BlockSpec((SBLK, H), lambda i: (i, 0)),
            pl.BlockSpec((1, E, H), lambda i: (0, 0, 0)),
            pl.BlockSpec((1, H, E), lambda i: (0, 0, 0)),
        ],
        out_specs=pl.BlockSpec((SBLK, H), lambda i: (i, 0)),
        out_shape=jax.ShapeDtypeStruct((T, H), jnp.float32),
        interpret=interpret,
    )(x, su, sd)


def _run_gmm(sp, xs, ru, rd, *, interpret=False):
    ridx = lambda i, sp: (jnp.minimum(sp[1 + i], NR - 1), 0, 0)
    grid_spec = pltpu.PrefetchScalarGridSpec(
        num_scalar_prefetch=1,
        grid=(NTAB,),
        in_specs=[
            pl.BlockSpec((BLK, H),
                         lambda i, sp: (jnp.minimum(i, sp[0] - 1), 0)),
            pl.BlockSpec((1, E, H), ridx),
            pl.BlockSpec((1, H, E), ridx),
        ],
        out_specs=pl.BlockSpec(
            (BLK, H), lambda i, sp: (jnp.minimum(i, sp[0] - 1), 0)),
    )
    return pl.pallas_call(
        _gmm_body,
        grid_spec=grid_spec,
        out_shape=jax.ShapeDtypeStruct((NROW, H), jnp.float32),
        compiler_params=pltpu.CompilerParams(
            dimension_semantics=("arbitrary",)),
        interpret=interpret,
    )(sp, xs, ru, rd)


def _combine_chunk_start(y_hbm, ys_hbm, sidx_hbm, gp0_hbm, gp1_hbm,
                         bufs, idxs, sems, wid, c):
    tb = wid * TPW + c * CH
    p = c % 2
    j = (wid * NCH + c) * 2
    acc, b0, b1, g0, g1 = bufs[p]
    i0, i1 = idxs[p]
    pltpu.sync_copy(sidx_hbm.at[j], i0)
    pltpu.sync_copy(sidx_hbm.at[j + 1], i1)
    copies = (pltpu.async_copy(ys_hbm.at[pl.ds(tb, CH)], acc, sems[p]),
              pltpu.async_copy(y_hbm.at[i0], b0, sems[p]),
              pltpu.async_copy(y_hbm.at[i1], b1, sems[p]),
              pltpu.async_copy(gp0_hbm.at[pl.ds(tb, CH)], g0, sems[p]),
              pltpu.async_copy(gp1_hbm.at[pl.ds(tb, CH)], g1, sems[p]))
    return copies


def _combine_body(y_hbm, ys_hbm, sidx_hbm, gp0_hbm, gp1_hbm, out_hbm,
                  acc0, b00, b10, g00, g10, acc1, b01, b11, g01, g11,
                  i00, i10, i01, i11, sem0, sem1):
    wid = lax.axis_index("s") * 2 + lax.axis_index("c")
    bufs = ((acc0, b00, b10, g00, g10), (acc1, b01, b11, g01, g11))
    idxs = ((i00, i10), (i01, i11))
    sems = (sem0, sem1)
    pend = _combine_chunk_start(y_hbm, ys_hbm, sidx_hbm, gp0_hbm, gp1_hbm,
                                bufs, idxs, sems, wid, 0)
    for c in range(NCH):
        if c + 1 < NCH:
            nxt = _combine_chunk_start(y_hbm, ys_hbm, sidx_hbm, gp0_hbm,
                                       gp1_hbm, bufs, idxs, sems, wid, c + 1)
        for cp in pend:
            cp.wait()
        p = c % 2
        acc, b0, b1, g0, g1 = bufs[p]

        def _row(r, carry, acc=acc, b0=b0, b1=b1, g0=g0, g1=g1):
            g0r = g0[r, :]
            g1r = g1[r, :]

            @plsc.parallel_loop(0, H, step=16, unroll=8)
            def _add(col):
                acc[r, pl.ds(col, 16)] = (acc[r, pl.ds(col, 16)]
                                          + g0r * b0[r, pl.ds(col, 16)]
                                          + g1r * b1[r, pl.ds(col, 16)])
            return carry
        lax.fori_loop(0, CH, _row, 0)
        tb = wid * TPW + c * CH
        pltpu.sync_copy(acc, out_hbm.at[pl.ds(tb, CH)])
        if c + 1 < NCH:
            pend = nxt


def _run_combine(yg, ys, sidx, gp0, gp1, *, interpret=False):
    mesh = plsc.VectorSubcoreMesh(core_axis_name="c", subcore_axis_name="s")
    f = pl.kernel(
        _combine_body,
        out_type=jax.ShapeDtypeStruct((T, H), jnp.float32),
        mesh=mesh,
        scratch_types=(
            [pltpu.VMEM((CH, H), jnp.float32)] * 3
            + [pltpu.VMEM((CH, 16), jnp.float32)] * 2
            + [pltpu.VMEM((CH, H), jnp.float32)] * 3
            + [pltpu.VMEM((CH, 16), jnp.float32)] * 2
            + [pltpu.VMEM((CH,), jnp.int32)] * 4
            + [pltpu.SemaphoreType.DMA, pltpu.SemaphoreType.DMA]
        ),
        interpret=interpret,
    )
    return f(yg, ys, sidx, gp0, gp1)


def _tile_layout(v2048):
    return v2048.reshape(NW, NCH, CH)


def kernel(x, shared_up, shared_down, routed_up, routed_down, router_W):
    s0, s1, gp0, gp1, be, nbt = _run_route_meta(x, router_W)
    sp = jnp.concatenate([nbt.reshape(1), be.reshape(2 * BLK)[:NTAB]])
    s0t = _tile_layout(s0.reshape(T))
    s1t = _tile_layout(s1.reshape(T))
    sidx = jnp.stack([s0t, s1t], axis=2).reshape(NW * NCH * 2, CH)
    xs = _run_dispatch(x, sidx)
    ys = _run_shared(x, shared_up, shared_down)
    yg = _run_gmm(sp, xs, routed_up, routed_down)
    return _run_combine(yg, ys, sidx, gp0, gp1)


# speedup vs baseline: 1.1620x; 1.1620x over previous
import functools

import jax
import jax.numpy as jnp
from jax import lax
from jax.experimental import pallas as pl
from jax.experimental.pallas import tpu as pltpu
from jax.experimental.pallas import tpu_sc as plsc

T = 2048
H = 1024
E = 512
NR = 64
NE = NR + 1
TB = 256
NB = T // TB
BLK = 128
NRB = 96
NTAB = NRB
NROW = NRB * BLK
NW = 32
TPW = T // NW
CH = 16
NCH = TPW // CH


def _route_meta_body(x_ref, rw_ref,
                     s0_ref, s1_ref, g0_ref, g1_ref, be_ref, nbt_ref,
                     i1_s, i2_s, r0_s, r1_s, c0_s, c1_s):
    i = pl.program_id(0)

    @pl.when(i < NB)
    def _router():
        x = x_ref[...]
        logits = lax.dot_general(x, rw_ref[...], (((1,), (1,)), ((), ())),
                                 preferred_element_type=jnp.float32)
        m = jnp.max(logits, axis=-1, keepdims=True)
        ex = jnp.exp(logits - m)
        probs = ex / jnp.sum(ex, axis=-1, keepdims=True)
        idx = lax.broadcasted_iota(jnp.int32, probs.shape, 1)
        m1 = jnp.max(probs, axis=-1, keepdims=True)
        i1 = jnp.min(jnp.where(probs == m1, idx, NR), axis=-1, keepdims=True)
        masked = jnp.where(idx == i1, -1.0, probs)
        m2 = jnp.max(masked, axis=-1, keepdims=True)
        i2 = jnp.min(jnp.where(masked == m2, idx, NR), axis=-1,
                     keepdims=True)
        off = i * TB
        g0_ref[pl.ds(off, TB), :] = jnp.broadcast_to(m1, (TB, 16))
        g1_ref[pl.ds(off, TB), :] = jnp.broadcast_to(m2, (TB, 16))
        i1_s[pl.ds(off, TB), :] = i1
        i2_s[pl.ds(off, TB), :] = i2
        o1 = (i1 == idx).astype(jnp.float32)
        o2 = (i2 == idx).astype(jnp.float32)
        r_i = lax.broadcasted_iota(jnp.int32, (TB, TB), 0)
        c_i = lax.broadcasted_iota(jnp.int32, (TB, TB), 1)
        tril = (r_i > c_i).astype(jnp.float32)
        cum1 = lax.dot_general(tril, o1, (((1,), (0,)), ((), ())),
                               preferred_element_type=jnp.float32)
        cum2 = lax.dot_general(tril, o2, (((1,), (0,)), ((), ())),
                               preferred_element_type=jnp.float32)
        r0_s[pl.ds(off, TB), :] = jnp.sum(
            cum1 * o1, axis=-1, keepdims=True).astype(jnp.int32)
        r1_s[pl.ds(off, TB), :] = jnp.sum(
            cum2 * o2, axis=-1, keepdims=True).astype(jnp.int32)
        rowsel = lax.broadcasted_iota(jnp.int32, (NB, NR), 0) == i
        c0_s[...] = jnp.where(rowsel, jnp.sum(o1, axis=0, keepdims=True),
                              c0_s[...])
        c1_s[...] = jnp.where(rowsel, jnp.sum(o2, axis=0, keepdims=True),
                              c1_s[...])

    @pl.when(i == NB)
    def _meta():
        c0 = c0_s[...]
        c1 = c1_s[...]
        r8 = lax.broadcasted_iota(jnp.int32, (NB, NB), 0)
        c8 = lax.broadcasted_iota(jnp.int32, (NB, NB), 1)
        tril8 = (r8 > c8).astype(jnp.float32)
        carry0 = lax.dot_general(tril8, c0, (((1,), (0,)), ((), ())),
                                 preferred_element_type=jnp.float32)
        carry1 = lax.dot_general(tril8, c1, (((1,), (0,)), ((), ())),
                                 preferred_element_type=jnp.float32)
        cnt0 = jnp.sum(c0, axis=0, keepdims=True)
        cnt1 = jnp.sum(c1, axis=0, keepdims=True)
        ctot = (cnt0 + cnt1).astype(jnp.int32)
        nb = (ctot + (BLK - 1)) // BLK
        nbf = nb.astype(jnp.float32)
        rE = lax.broadcasted_iota(jnp.int32, (NR, NR), 0)
        cE = lax.broadcasted_iota(jnp.int32, (NR, NR), 1)
        triu = (rE < cE).astype(jnp.float32)
        bc_excl = lax.dot_general(nbf, triu, (((1,), (0,)), ((), ())),
                                  preferred_element_type=jnp.float32)
        bc_incl = bc_excl + nbf
        nbt_ref[...] = jnp.sum(nb, axis=-1, keepdims=True)
        tb_of = lax.broadcasted_iota(jnp.int32, (T, NB), 0) // TB
        sel = (tb_of == lax.broadcasted_iota(jnp.int32, (T, NB), 1)).astype(
            jnp.float32)
        car0 = lax.dot_general(sel, carry0, (((1,), (0,)), ((), ())),
                               preferred_element_type=jnp.float32)
        car1 = lax.dot_general(sel, carry1, (((1,), (0,)), ((), ())),
                               preferred_element_type=jnp.float32)
        ie = lax.broadcasted_iota(jnp.int32, (T, NR), 1)
        o1 = (i1_s[...] == ie).astype(jnp.float32)
        o2 = (i2_s[...] == ie).astype(jnp.float32)
        base0 = jnp.sum(o1 * (BLK * bc_excl + car0), axis=-1, keepdims=True)
        base1 = jnp.sum(o2 * (BLK * bc_excl + cnt0 + car1), axis=-1,
                        keepdims=True)
        s0_ref[...] = base0.astype(jnp.int32) + r0_s[...]
        s1_ref[...] = base1.astype(jnp.int32) + r1_s[...]
        bi = lax.broadcasted_iota(jnp.int32, (2 * BLK, NR), 0).astype(
            jnp.float32)
        be = jnp.sum((bi >= bc_incl).astype(jnp.float32), axis=-1,
                     keepdims=True)
        be_ref[...] = be.astype(jnp.int32)


def _run_route_meta(x, router_W, *, interpret=False):
    full = lambda i: (0, 0)
    return pl.pallas_call(
        _route_meta_body,
        grid=(NB + 1,),
        in_specs=[
            pl.BlockSpec((TB, H), lambda i: (jnp.minimum(i, NB - 1), 0)),
            pl.BlockSpec((NR, H), full),
        ],
        out_specs=[
            pl.BlockSpec((T, 1), full),
            pl.BlockSpec((T, 1), full),
            pl.BlockSpec((T, 16), full),
            pl.BlockSpec((T, 16), full),
            pl.BlockSpec((2 * BLK, 1), full),
            pl.BlockSpec((1, 1), full),
        ],
        out_shape=[
            jax.ShapeDtypeStruct((T, 1), jnp.int32),
            jax.ShapeDtypeStruct((T, 1), jnp.int32),
            jax.ShapeDtypeStruct((T, 16), jnp.float32),
            jax.ShapeDtypeStruct((T, 16), jnp.float32),
            jax.ShapeDtypeStruct((2 * BLK, 1), jnp.int32),
            jax.ShapeDtypeStruct((1, 1), jnp.int32),
        ],
        scratch_shapes=[
            pltpu.VMEM((T, 1), jnp.int32),
            pltpu.VMEM((T, 1), jnp.int32),
            pltpu.VMEM((T, 1), jnp.int32),
            pltpu.VMEM((T, 1), jnp.int32),
            pltpu.VMEM((NB, NR), jnp.float32),
            pltpu.VMEM((NB, NR), jnp.float32),
        ],
        interpret=interpret,
    )(x, router_W)


def _dispatch_body(x_hbm, sidx_hbm, xs_hbm,
                   rbuf0, rbuf1, idx0, idx1, lsem, sem):
    wid = lax.axis_index("s") * 2 + lax.axis_index("c")
    rbufs = (rbuf0, rbuf1)
    idxs = (idx0, idx1)
    base = wid * TPW
    ld0 = pltpu.async_copy(x_hbm.at[pl.ds(base, CH)], rbuf0, lsem)
    loads = [ld0]
    for c in range(NCH):
        if c + 1 < NCH:
            loads.append(pltpu.async_copy(
                x_hbm.at[pl.ds(base + (c + 1) * CH, CH)],
                rbufs[(c + 1) % 2], lsem))
        j = (wid * NCH + c) * 2
        pltpu.sync_copy(sidx_hbm.at[j], idxs[0])
        pltpu.sync_copy(sidx_hbm.at[j + 1], idxs[1])
        loads[c].wait()
        s0 = pltpu.async_copy(rbufs[c % 2], xs_hbm.at[idxs[0]], sem)
        s1 = pltpu.async_copy(rbufs[c % 2], xs_hbm.at[idxs[1]], sem)
        s0.wait()
        s1.wait()


def _run_dispatch(x, sidx, *, interpret=False):
    mesh = plsc.VectorSubcoreMesh(core_axis_name="c", subcore_axis_name="s")
    f = pl.kernel(
        _dispatch_body,
        out_type=jax.ShapeDtypeStruct((NROW, H), jnp.float32),
        mesh=mesh,
        scratch_types=[
            pltpu.VMEM((CH, H), jnp.float32),
            pltpu.VMEM((CH, H), jnp.float32),
            pltpu.VMEM((CH,), jnp.int32),
            pltpu.VMEM((CH,), jnp.int32),
            pltpu.SemaphoreType.DMA,
            pltpu.SemaphoreType.DMA,
        ],
        interpret=interpret,
    )
    return f(x, sidx)


def _gelu(u):
    return 0.5 * u * (1.0 + lax.erf(u * (2.0 ** -0.5)))


def _expert_block(xb, wu, wd):
    up = lax.dot_general(xb.astype(jnp.bfloat16), wu.astype(jnp.bfloat16),
                         (((1,), (1,)), ((), ())),
                         preferred_element_type=jnp.float32)
    act = _gelu(up)
    return lax.dot_general(act.astype(jnp.bfloat16),
                           wd.astype(jnp.bfloat16),
                           (((1,), (1,)), ((), ())),
                           preferred_element_type=jnp.float32)


def _gmm_body(sp_ref, xs_ref, ru_ref, rd_ref, out_ref):
    i = pl.program_id(0)

    @pl.when(i < sp_ref[0])
    def _():
        out_ref[...] = _expert_block(xs_ref[...], ru_ref[0], rd_ref[0])


def _shared_body(x_ref, su_ref, sd_ref, out_ref):
    out_ref[...] = _expert_block(x_ref[...], su_ref[0], sd_ref[0])


def _run_shared(x, su, sd, *, interpret=False):
    return pl.pallas_call(
        _shared_body,
        grid=(T // BLK,),
        in_specs=[
            pl.BlockSpec((BLK, H), lambda i: (i, 0)),
            pl.BlockSpec((1, E, H), lambda i: (0, 0, 0)),
            pl.BlockSpec((1, H, E), lambda i: (0, 0, 0)),
        ],
        out_specs=pl.BlockSpec((BLK, H), lambda i: (i, 0)),
        out_shape=jax.ShapeDtypeStruct((T, H), jnp.float32),
        interpret=interpret,
    )(x, su, sd)


def _run_gmm(sp, xs, ru, rd, *, interpret=False):
    ridx = lambda i, sp: (jnp.minimum(sp[1 + i], NR - 1), 0, 0)
    grid_spec = pltpu.PrefetchScalarGridSpec(
        num_scalar_prefetch=1,
        grid=(NTAB,),
        in_specs=[
            pl.BlockSpec((BLK, H),
                         lambda i, sp: (jnp.minimum(i, sp[0] - 1), 0)),
            pl.BlockSpec((1, E, H), ridx),
            pl.BlockSpec((1, H, E), ridx),
        ],
        out_specs=pl.BlockSpec(
            (BLK, H), lambda i, sp: (jnp.minimum(i, sp[0] - 1), 0)),
    )
    return pl.pallas_call(
        _gmm_body,
        grid_spec=grid_spec,
        out_shape=jax.ShapeDtypeStruct((NROW, H), jnp.float32),
        compiler_params=pltpu.CompilerParams(
            dimension_semantics=("arbitrary",)),
        interpret=interpret,
    )(sp, xs, ru, rd)


def _combine_chunk_start(y_hbm, ys_hbm, sidx_hbm, gp0_hbm, gp1_hbm,
                         bufs, idxs, sems, wid, c):
    tb = wid * TPW + c * CH
    p = c % 2
    j = (wid * NCH + c) * 2
    acc, b0, b1, g0, g1 = bufs[p]
    i0, i1 = idxs[p]
    pltpu.sync_copy(sidx_hbm.at[j], i0)
    pltpu.sync_copy(sidx_hbm.at[j + 1], i1)
    copies = (pltpu.async_copy(ys_hbm.at[pl.ds(tb, CH)], acc, sems[p]),
              pltpu.async_copy(y_hbm.at[i0], b0, sems[p]),
              pltpu.async_copy(y_hbm.at[i1], b1, sems[p]),
              pltpu.async_copy(gp0_hbm.at[pl.ds(tb, CH)], g0, sems[p]),
              pltpu.async_copy(gp1_hbm.at[pl.ds(tb, CH)], g1, sems[p]))
    return copies


def _combine_body(y_hbm, ys_hbm, sidx_hbm, gp0_hbm, gp1_hbm, out_hbm,
                  acc0, b00, b10, g00, g10, acc1, b01, b11, g01, g11,
                  i00, i10, i01, i11, sem0, sem1):
    wid = lax.axis_index("s") * 2 + lax.axis_index("c")
    bufs = ((acc0, b00, b10, g00, g10), (acc1, b01, b11, g01, g11))
    idxs = ((i00, i10), (i01, i11))
    sems = (sem0, sem1)
    pend = _combine_chunk_start(y_hbm, ys_hbm, sidx_hbm, gp0_hbm, gp1_hbm,
                                bufs, idxs, sems, wid, 0)
    for c in range(NCH):
        if c + 1 < NCH:
            nxt = _combine_chunk_start(y_hbm, ys_hbm, sidx_hbm, gp0_hbm,
                                       gp1_hbm, bufs, idxs, sems, wid, c + 1)
        for cp in pend:
            cp.wait()
        p = c % 2
        acc, b0, b1, g0, g1 = bufs[p]

        def _row(r, carry, acc=acc, b0=b0, b1=b1, g0=g0, g1=g1):
            g0r = g0[r, :]
            g1r = g1[r, :]

            @plsc.parallel_loop(0, H, step=16, unroll=8)
            def _add(col):
                acc[r, pl.ds(col, 16)] = (acc[r, pl.ds(col, 16)]
                                          + g0r * b0[r, pl.ds(col, 16)]
                                          + g1r * b1[r, pl.ds(col, 16)])
            return carry
        lax.fori_loop(0, CH, _row, 0)
        tb = wid * TPW + c * CH
        pltpu.sync_copy(acc, out_hbm.at[pl.ds(tb, CH)])
        if c + 1 < NCH:
            pend = nxt


def _run_combine(yg, ys, sidx, gp0, gp1, *, interpret=False):
    mesh = plsc.VectorSubcoreMesh(core_axis_name="c", subcore_axis_name="s")
    f = pl.kernel(
        _combine_body,
        out_type=jax.ShapeDtypeStruct((T, H), jnp.float32),
        mesh=mesh,
        scratch_types=(
            [pltpu.VMEM((CH, H), jnp.float32)] * 3
            + [pltpu.VMEM((CH, 16), jnp.float32)] * 2
            + [pltpu.VMEM((CH, H), jnp.float32)] * 3
            + [pltpu.VMEM((CH, 16), jnp.float32)] * 2
            + [pltpu.VMEM((CH,), jnp.int32)] * 4
            + [pltpu.SemaphoreType.DMA, pltpu.SemaphoreType.DMA]
        ),
        interpret=interpret,
    )
    return f(yg, ys, sidx, gp0, gp1)


def _tile_layout(v2048):
    return v2048.reshape(NW, NCH, CH)


def kernel(x, shared_up, shared_down, routed_up, routed_down, router_W):
    s0, s1, gp0, gp1, be, nbt = _run_route_meta(x, router_W)
    sp = jnp.concatenate([nbt.reshape(1), be.reshape(2 * BLK)[:NTAB]])
    s0t = _tile_layout(s0.reshape(T))
    s1t = _tile_layout(s1.reshape(T))
    sidx = jnp.stack([s0t, s1t], axis=2).reshape(NW * NCH * 2, CH)
    xs = _run_dispatch(x, sidx)
    ys = _run_shared(x, shared_up, shared_down)
    yg = _run_gmm(sp, xs, routed_up, routed_down)
    return _run_combine(yg, ys, sidx, gp0, gp1)
